# baseline (speedup 1.0000x reference)
import functools

import jax
import jax.numpy as jnp
from jax import lax
from jax.experimental import pallas as pl
from jax.experimental.pallas import tpu as pltpu
from jax.experimental.pallas import tpu_sc as plsc

NUM_PROMPTS = 100000
NUM_TOKENS = 20
TOKEN_DIM = 32
BATCH = 4096
D = NUM_TOKENS * TOKEN_DIM

NC = 2
NS = 16
NW = NC * NS
P_PER_W = NUM_PROMPTS // NW
LANES = 16

NTROW = D // 8
SLAB_TILES = 26
SLAB_W = SLAB_TILES * 128
COL0_MAX = 755
TAIL_COL = 99968
TAIL_W = NUM_PROMPTS - TAIL_COL

CAP = 192
NCHUNK = CAP // LANES
CAPX = CAP + LANES
OUT_ROWS = BATCH + 8
B_SCAN = BATCH // LANES


def _sc_body(tbl, idx_hbm, x_hbm, out_hbm, stage_hbm,
             idx_v, blist_f, clist_f, plist_f, blist2,
             fbuf, slab, tailb, gbuf, xbuf, rbuf,
             sem_slab, sem_tail, sem_fb, sem_g, sem_x, sem_o):
    cid = lax.axis_index("c")
    sid = lax.axis_index("s")
    wid = sid * NC + cid
    lo = wid * P_PER_W
    hi = lo + P_PER_W
    col0 = jnp.minimum(lo // 128, COL0_MAX)
    c_base = col0 * 128
    is_last = wid == NW - 1

    pltpu.sync_copy(idx_hbm.at[pl.ds(0, BATCH)], idx_v)

    pad_b = jnp.full((LANES,), BATCH, jnp.int32)
    zero16 = jnp.zeros((LANES,), jnp.int32)
    for k in range(NCHUNK + 1):
        blist_f[pl.ds(k * LANES, LANES)] = pad_b
        clist_f[pl.ds(k * LANES, LANES)] = zero16
        plist_f[pl.ds(k * LANES, LANES)] = zero16

    lane = lax.iota(jnp.int32, LANES)

    def scan_body(j, cnt):
        v = idx_v[pl.ds(j * LANES, LANES)]
        m = (v >= lo) & (v < hi)
        b_ids = lane + j * LANES
        c_loc = jnp.minimum(v, TAIL_COL - 1) - c_base
        off = jnp.minimum(cnt, CAP - LANES)
        cum = plsc.cumsum(m.astype(jnp.int32))
        pos = jnp.where(m, off + cum - 1, CAP + lane)
        plsc.store_scatter(blist_f, [pos], b_ids)
        plsc.store_scatter(clist_f, [pos], c_loc)
        plsc.store_scatter(plist_f, [pos], v)
        return cnt + cum[LANES - 1]

    lax.fori_loop(0, B_SCAN, scan_body, jnp.int32(0))

    for k in range(NCHUNK):
        blist2[k, pl.ds(0, LANES)] = blist_f[pl.ds(k * LANES, LANES)]

    def slab_copy(r, par):
        return pltpu.make_async_copy(
            tbl.at[pl.ds(8 * r, 8), pl.ds(c_base, SLAB_W)],
            slab.at[par], sem_slab.at[par])

    def tail_copy(r, par):
        return pltpu.make_async_copy(
            tbl.at[pl.ds(8 * r, 8), pl.ds(TAIL_COL, TAIL_W)],
            tailb.at[par], sem_tail.at[par])

    slab_copy(0, 0).start()
    slab_copy(1, 1).start()

    @pl.when(is_last)
    def _():
        tail_copy(0, 0).start()
        tail_copy(1, 1).start()

    def do_row(r, par):
        slab_copy(r, par).wait()

        @pl.when(r < NTROW - 2)
        def _():
            slab_copy(r + 2, (par + 2) % 3).start()

        @pl.when(is_last)
        def _():
            tail_copy(r, par).wait()

            @pl.when(r < NTROW - 2)
            def _():
                tail_copy(r + 2, (par + 2) % 3).start()

        def fb_copy(rr, fpar):
            return pltpu.make_async_copy(
                fbuf.at[fpar], stage_hbm.at[wid, :, rr], sem_fb.at[fpar])

        @pl.when(r >= 3)
        def _():
            fb_copy(r, par).wait()

        src = slab.at[par]
        fb = fbuf.at[par]
        for k in range(1):
            cols = clist_f[pl.ds(k * LANES, LANES)]
            for g in range(8):
                vals = plsc.load_gather(src, [jnp.full((LANES,), g, jnp.int32), cols])
                fb[k, pl.ds(g * LANES, LANES)] = vals

        @pl.when(is_last)
        def _():
            tsrc = tailb.at[par]
            for k in range(NCHUNK):
                pv = plist_f[pl.ds(k * LANES, LANES)]
                tm = pv >= TAIL_COL
                tcols = jnp.clip(pv - TAIL_COL, 0, TAIL_W - 1)
                for g in range(8):
                    tvals = plsc.load_gather(
                        tsrc, [jnp.full((LANES,), g, jnp.int32), tcols])
                    cur = fb[k, pl.ds(g * LANES, LANES)]
                    fb[k, pl.ds(g * LANES, LANES)] = jnp.where(tm, tvals, cur)

        fb_copy(r, par).start()

    def tri_body(i, carry):
        do_row(3 * i, 0)
        do_row(3 * i + 1, 1)
        do_row(3 * i + 2, 2)
        return carry

    lax.fori_loop(0, NTROW // 3, tri_body, jnp.int32(0))
    do_row(NTROW - 2, 0)
    do_row(NTROW - 1, 1)

    pltpu.make_async_copy(
        fbuf.at[2], stage_hbm.at[wid, :, NTROW - 3], sem_fb.at[2]).wait()
    pltpu.make_async_copy(
        fbuf.at[0], stage_hbm.at[wid, :, NTROW - 2], sem_fb.at[0]).wait()
    pltpu.make_async_copy(
        fbuf.at[1], stage_hbm.at[wid, :, NTROW - 1], sem_fb.at[1]).wait()

    outs = []
    for bc in range(NCHUNK):
        gc = pltpu.async_copy(stage_hbm.at[wid, bc], gbuf, sem_g.at[0])
        xc = pltpu.async_copy(x_hbm.at[blist2.at[bc]], xbuf, sem_x.at[0])
        gc.wait()
        xc.wait()
        if bc >= 1:
            outs[bc - 1].wait()

        gsrc = gbuf
        xsrc = xbuf
        rdst = rbuf

        def tr_body(tr, carry, gsrc=gsrc, xsrc=xsrc, rdst=rdst):
            for g in range(8):
                v = gsrc[tr, pl.ds(g * LANES, LANES)]
                fv = jnp.full((LANES,), g, jnp.int32) + tr * 8
                xv = plsc.load_gather(xsrc, [lane, fv])
                plsc.store_scatter(rdst, [lane, fv], v + xv)
            return carry

        lax.fori_loop(0, NTROW, tr_body, jnp.int32(0))

        oc = pltpu.async_copy(rdst, out_hbm.at[blist2.at[bc]], sem_o.at[0])
        outs.append(oc)

    outs[-1].wait()


@functools.partial(jax.jit, static_argnames=())
def kernel(x, idx, prompts):
    tbl_T = prompts.transpose(1, 2, 0).reshape(D, NUM_PROMPTS)
    x_pad = jnp.pad(x.reshape(BATCH, D).astype(jnp.float32),
                    ((0, OUT_ROWS - BATCH), (0, 0)))
    idx32 = idx.astype(jnp.int32)

    mesh = plsc.VectorSubcoreMesh(
        core_axis_name="c", subcore_axis_name="s",
        num_cores=NC, num_subcores=NS,
    )
    out_pad = pl.kernel(
        _sc_body,
        out_type=[jax.ShapeDtypeStruct((OUT_ROWS, D), jnp.float32),
                  jax.ShapeDtypeStruct((NW, NCHUNK, NTROW, 128), jnp.float32)],
        mesh=mesh,
        scratch_types=[
            pltpu.VMEM((BATCH,), jnp.int32),
            pltpu.VMEM((CAPX,), jnp.int32),
            pltpu.VMEM((CAPX,), jnp.int32),
            pltpu.VMEM((CAPX,), jnp.int32),
            pltpu.VMEM((NCHUNK, LANES), jnp.int32),
            pltpu.VMEM((3, NCHUNK, 8 * LANES), jnp.float32),
            pltpu.VMEM((3, 8, SLAB_W), jnp.float32),
            pltpu.VMEM((3, 8, TAIL_W), jnp.float32),
            pltpu.VMEM((NTROW, 8 * LANES), jnp.float32),
            pltpu.VMEM((LANES, D), jnp.float32),
            pltpu.VMEM((LANES, D), jnp.float32),
            pltpu.SemaphoreType.DMA((3,)),
            pltpu.SemaphoreType.DMA((3,)),
            pltpu.SemaphoreType.DMA((3,)),
            pltpu.SemaphoreType.DMA((2,)),
            pltpu.SemaphoreType.DMA((2,)),
            pltpu.SemaphoreType.DMA((2,)),
        ],
        compiler_params=pltpu.CompilerParams(needs_layout_passes=False),
    )(tbl_T, idx32, x_pad)
    out_pad = out_pad[0]
    return out_pad[:BATCH].reshape(BATCH, NUM_TOKENS, TOKEN_DIM)


# speedup vs baseline: 1.0173x; 1.0173x over previous
import functools

import jax
import jax.numpy as jnp
from jax import lax
from jax.experimental import pallas as pl
from jax.experimental.pallas import tpu as pltpu
from jax.experimental.pallas import tpu_sc as plsc

NUM_PROMPTS = 100000
NUM_TOKENS = 20
TOKEN_DIM = 32
BATCH = 4096
D = NUM_TOKENS * TOKEN_DIM

NC = 2
NS = 16
NW = NC * NS
P_PER_W = NUM_PROMPTS // NW
LANES = 16

NTROW = D // 8
SLAB_TILES = 26
SLAB_W = SLAB_TILES * 128
COL0_MAX = 755
TAIL_COL = 99968
TAIL_W = NUM_PROMPTS - TAIL_COL

CAP = 192
NCHUNK = CAP // LANES
CAPX = CAP + LANES
OUT_ROWS = BATCH + 8
B_SCAN = BATCH // LANES


def _sc_body(tbl, idx_hbm, x_hbm, out_hbm, stage_hbm,
             idx_v, blist_f, clist_f, plist_f, blist2,
             fbuf, slab, tailb, gbuf, xbuf, rbuf,
             sem_slab, sem_tail, sem_fb, sem_g, sem_x, sem_o):
    cid = lax.axis_index("c")
    sid = lax.axis_index("s")
    wid = sid * NC + cid
    lo = wid * P_PER_W
    hi = lo + P_PER_W
    col0 = jnp.minimum(lo // 128, COL0_MAX)
    c_base = col0 * 128
    is_last = wid == NW - 1

    pltpu.sync_copy(idx_hbm.at[pl.ds(0, BATCH)], idx_v)

    pad_b = jnp.full((LANES,), BATCH, jnp.int32)
    zero16 = jnp.zeros((LANES,), jnp.int32)
    for k in range(NCHUNK + 1):
        blist_f[pl.ds(k * LANES, LANES)] = pad_b
        clist_f[pl.ds(k * LANES, LANES)] = zero16
        plist_f[pl.ds(k * LANES, LANES)] = zero16

    lane = lax.iota(jnp.int32, LANES)

    def scan_body(j, cnt):
        v = idx_v[pl.ds(j * LANES, LANES)]
        m = (v >= lo) & (v < hi)
        b_ids = lane + j * LANES
        c_loc = jnp.minimum(v, TAIL_COL - 1) - c_base
        off = jnp.minimum(cnt, CAP - LANES)
        cum = plsc.cumsum(m.astype(jnp.int32))
        pos = jnp.where(m, off + cum - 1, CAP + lane)
        plsc.store_scatter(blist_f, [pos], b_ids)
        plsc.store_scatter(clist_f, [pos], c_loc)
        plsc.store_scatter(plist_f, [pos], v)
        return cnt + cum[LANES - 1]

    lax.fori_loop(0, B_SCAN, scan_body, jnp.int32(0))

    for k in range(NCHUNK):
        blist2[k, pl.ds(0, LANES)] = blist_f[pl.ds(k * LANES, LANES)]

    HALF = SLAB_W // 2

    def slab_copies(r, par):
        return [pltpu.make_async_copy(
            tbl.at[pl.ds(8 * r, 8), pl.ds(c_base + q * HALF, HALF)],
            slab.at[par, :, pl.ds(q * HALF, HALF)], sem_slab.at[par])
            for q in range(2)]

    def slab_start(r, par):
        for c in slab_copies(r, par):
            c.start()

    def slab_wait(r, par):
        for c in slab_copies(r, par):
            c.wait()

    def tail_copy(r, par):
        return pltpu.make_async_copy(
            tbl.at[pl.ds(8 * r, 8), pl.ds(TAIL_COL, TAIL_W)],
            tailb.at[par], sem_tail.at[par])

    slab_start(0, 0)
    slab_start(1, 1)

    @pl.when(is_last)
    def _():
        tail_copy(0, 0).start()
        tail_copy(1, 1).start()

    def do_row(r, par):
        @pl.when(r < NTROW - 2)
        def _():
            slab_start(r + 2, (par + 2) % 3)

        slab_wait(r, par)

        @pl.when(is_last)
        def _():
            @pl.when(r < NTROW - 2)
            def _():
                tail_copy(r + 2, (par + 2) % 3).start()

            tail_copy(r, par).wait()

        def fb_copy(rr, fpar):
            return pltpu.make_async_copy(
                fbuf.at[fpar], stage_hbm.at[wid, :, rr], sem_fb.at[fpar])

        @pl.when(r >= 3)
        def _():
            fb_copy(r, par).wait()

        src = slab.at[par]
        fb = fbuf.at[par]
        for k in range(NCHUNK):
            cols = clist_f[pl.ds(k * LANES, LANES)]
            for g in range(8):
                vals = plsc.load_gather(src, [jnp.full((LANES,), g, jnp.int32), cols])
                fb[k, pl.ds(g * LANES, LANES)] = vals

        @pl.when(is_last)
        def _():
            tsrc = tailb.at[par]
            for k in range(NCHUNK):
                pv = plist_f[pl.ds(k * LANES, LANES)]
                tm = pv >= TAIL_COL
                tcols = jnp.clip(pv - TAIL_COL, 0, TAIL_W - 1)
                for g in range(8):
                    tvals = plsc.load_gather(
                        tsrc, [jnp.full((LANES,), g, jnp.int32), tcols])
                    cur = fb[k, pl.ds(g * LANES, LANES)]
                    fb[k, pl.ds(g * LANES, LANES)] = jnp.where(tm, tvals, cur)

        fb_copy(r, par).start()

    def tri_body(i, carry):
        do_row(3 * i, 0)
        do_row(3 * i + 1, 1)
        do_row(3 * i + 2, 2)
        return carry

    lax.fori_loop(0, NTROW // 3, tri_body, jnp.int32(0))
    do_row(NTROW - 2, 0)
    do_row(NTROW - 1, 1)

    pltpu.make_async_copy(
        fbuf.at[2], stage_hbm.at[wid, :, NTROW - 3], sem_fb.at[2]).wait()
    pltpu.make_async_copy(
        fbuf.at[0], stage_hbm.at[wid, :, NTROW - 2], sem_fb.at[0]).wait()
    pltpu.make_async_copy(
        fbuf.at[1], stage_hbm.at[wid, :, NTROW - 1], sem_fb.at[1]).wait()

    outs = []
    for bc in range(NCHUNK):
        gc = pltpu.async_copy(stage_hbm.at[wid, bc], gbuf, sem_g.at[0])
        xc = pltpu.async_copy(x_hbm.at[blist2.at[bc]], xbuf, sem_x.at[0])
        gc.wait()
        xc.wait()
        if bc >= 1:
            outs[bc - 1].wait()

        gsrc = gbuf
        xsrc = xbuf
        rdst = rbuf

        def tr_body(tr, carry, gsrc=gsrc, xsrc=xsrc, rdst=rdst):
            for g in range(8):
                v = gsrc[tr, pl.ds(g * LANES, LANES)]
                fv = jnp.full((LANES,), g, jnp.int32) + tr * 8
                xv = plsc.load_gather(xsrc, [lane, fv])
                plsc.store_scatter(rdst, [lane, fv], v + xv)
            return carry

        lax.fori_loop(0, NTROW, tr_body, jnp.int32(0))

        oc = pltpu.async_copy(rdst, out_hbm.at[blist2.at[bc]], sem_o.at[0])
        outs.append(oc)

    outs[-1].wait()


@functools.partial(jax.jit, static_argnames=())
def kernel(x, idx, prompts):
    tbl_T = prompts.transpose(1, 2, 0).reshape(D, NUM_PROMPTS)
    x_pad = jnp.pad(x.reshape(BATCH, D).astype(jnp.float32),
                    ((0, OUT_ROWS - BATCH), (0, 0)))
    idx32 = idx.astype(jnp.int32)

    mesh = plsc.VectorSubcoreMesh(
        core_axis_name="c", subcore_axis_name="s",
        num_cores=NC, num_subcores=NS,
    )
    out_pad = pl.kernel(
        _sc_body,
        out_type=[jax.ShapeDtypeStruct((OUT_ROWS, D), jnp.float32),
                  jax.ShapeDtypeStruct((NW, NCHUNK, NTROW, 128), jnp.float32)],
        mesh=mesh,
        scratch_types=[
            pltpu.VMEM((BATCH,), jnp.int32),
            pltpu.VMEM((CAPX,), jnp.int32),
            pltpu.VMEM((CAPX,), jnp.int32),
            pltpu.VMEM((CAPX,), jnp.int32),
            pltpu.VMEM((NCHUNK, LANES), jnp.int32),
            pltpu.VMEM((3, NCHUNK, 8 * LANES), jnp.float32),
            pltpu.VMEM((3, 8, SLAB_W), jnp.float32),
            pltpu.VMEM((3, 8, TAIL_W), jnp.float32),
            pltpu.VMEM((NTROW, 8 * LANES), jnp.float32),
            pltpu.VMEM((LANES, D), jnp.float32),
            pltpu.VMEM((LANES, D), jnp.float32),
            pltpu.SemaphoreType.DMA((3,)),
            pltpu.SemaphoreType.DMA((3,)),
            pltpu.SemaphoreType.DMA((3,)),
            pltpu.SemaphoreType.DMA((2,)),
            pltpu.SemaphoreType.DMA((2,)),
            pltpu.SemaphoreType.DMA((2,)),
        ],
        compiler_params=pltpu.CompilerParams(needs_layout_passes=False),
    )(tbl_T, idx32, x_pad)
    out_pad = out_pad[0]
    return out_pad[:BATCH].reshape(BATCH, NUM_TOKENS, TOKEN_DIM)


# speedup vs baseline: 1.3513x; 1.3282x over previous
import functools

import jax
import jax.numpy as jnp
from jax import lax
from jax.experimental import pallas as pl
from jax.experimental.pallas import tpu as pltpu
from jax.experimental.pallas import tpu_sc as plsc

NUM_PROMPTS = 100000
NUM_TOKENS = 20
TOKEN_DIM = 32
BATCH = 4096
D = NUM_TOKENS * TOKEN_DIM

NC = 2
NS = 16
NW = NC * NS
B_PER_W = BATCH // NW
CHUNK = 32
N_CHUNKS = B_PER_W // CHUNK
LANES = 16
VECS_PER_ROW = D // LANES


def _sc_body(table_hbm, idx_hbm, x_hbm, out_hbm,
             idx_v, rows_v, x_v, gsems, xsems, osem):
    wid = lax.axis_index("s") * NC + lax.axis_index("c")
    base = wid * B_PER_W

    pltpu.sync_copy(idx_hbm.at[pl.ds(base, B_PER_W)], idx_v)

    def start_chunk(i):
        rows_dst = rows_v.at[pl.ds(i * CHUNK, CHUNK)]
        g = pltpu.async_copy(
            table_hbm.at[idx_v.at[pl.ds(i * CHUNK, CHUNK)]],
            rows_dst, gsems.at[i % 2])
        xc = pltpu.async_copy(
            x_hbm.at[pl.ds(base + i * CHUNK, CHUNK)],
            x_v.at[i % 2], xsems.at[i % 2])
        return g, xc

    pend = start_chunk(0)
    out_copies = []
    for i in range(N_CHUNKS):
        nxt = start_chunk(i + 1) if i + 1 < N_CHUNKS else None
        g, xc = pend
        g.wait()
        xc.wait()

        r0 = i * CHUNK
        xb = i % 2

        def add_row(r, _, r0=r0, xb=xb):
            for c in range(VECS_PER_ROW):
                col = pl.ds(c * LANES, LANES)
                rows_v[r0 + r, col] = rows_v[r0 + r, col] + x_v[xb, r, col]
            return 0

        lax.fori_loop(0, CHUNK, add_row, 0)

        oc = pltpu.async_copy(
            rows_v.at[pl.ds(r0, CHUNK)],
            out_hbm.at[pl.ds(base + r0, CHUNK)], osem)
        out_copies.append(oc)
        pend = nxt

    for oc in out_copies:
        oc.wait()


@functools.partial(jax.jit, static_argnames=())
def kernel(x, idx, prompts):
    table = prompts.reshape(NUM_PROMPTS, D)
    x2 = x.reshape(BATCH, D)
    idx32 = idx.astype(jnp.int32)

    mesh = plsc.VectorSubcoreMesh(
        core_axis_name="c", subcore_axis_name="s",
        num_cores=NC, num_subcores=NS,
    )
    out = pl.kernel(
        _sc_body,
        out_type=jax.ShapeDtypeStruct((BATCH, D), jnp.float32),
        mesh=mesh,
        scratch_types=[
            pltpu.VMEM((B_PER_W,), jnp.int32),
            pltpu.VMEM((B_PER_W, D), jnp.float32),
            pltpu.VMEM((2, CHUNK, D), jnp.float32),
            pltpu.SemaphoreType.DMA((2,)),
            pltpu.SemaphoreType.DMA((2,)),
            pltpu.SemaphoreType.DMA,
        ],
    )(table, idx32, x2)
    return out.reshape(BATCH, NUM_TOKENS, TOKEN_DIM)
